# baseline (speedup 1.0000x reference)
import jax
import jax.numpy as jnp
from jax.experimental import pallas as pl
from jax.experimental.pallas import tpu as pltpu

_T = 256
_B = 1024
_C = 192
_REMAIN = 64


def _shuffle_kernel(p_hbm, out_hbm, idx_ref, sem):
    for t in range(_REMAIN):
        pltpu.make_async_copy(p_hbm.at[_T - 1 - t], out_hbm.at[t], sem).start()
    idx_ref[...] = (_T - 1) - jax.lax.broadcasted_iota(jnp.int32, (_T, _B), 0)
    for t in range(_REMAIN):
        pltpu.make_async_copy(p_hbm.at[_T - 1 - t], out_hbm.at[t], sem).wait()


def kernel(patches):
    out, idx = pl.pallas_call(
        _shuffle_kernel,
        in_specs=[pl.BlockSpec(memory_space=pl.ANY)],
        out_specs=[
            pl.BlockSpec(memory_space=pl.ANY),
            pl.BlockSpec((_T, _B), lambda: (0, 0)),
        ],
        out_shape=[
            jax.ShapeDtypeStruct((_REMAIN, _B, _C), patches.dtype),
            jax.ShapeDtypeStruct((_T, _B), jnp.int32),
        ],
        scratch_shapes=[pltpu.SemaphoreType.DMA],
    )(patches)
    return (out, idx, idx)


# speedup vs baseline: 4.6344x; 4.6344x over previous
import jax
import jax.numpy as jnp
from jax.experimental import pallas as pl

_T = 256
_B = 1024
_C = 192
_REMAIN = 64
_S = _B * _C // 128


def _shuffle_kernel(p_ref, out_ref, idx_ref):
    i = pl.program_id(0)
    out_ref[...] = p_ref[...]
    row = i * 4 + jax.lax.broadcasted_iota(jnp.int32, (1, 4, _B), 1)
    idx_ref[...] = (_T - 1) - row


def kernel(patches):
    p = patches.reshape(_T, _S, 128)
    out, idx = pl.pallas_call(
        _shuffle_kernel,
        grid=(_REMAIN,),
        in_specs=[
            pl.BlockSpec((1, _S, 128), lambda i: (_T - 1 - i, 0, 0)),
        ],
        out_specs=[
            pl.BlockSpec((1, _S, 128), lambda i: (i, 0, 0)),
            pl.BlockSpec((1, 4, _B), lambda i: (i, 0, 0)),
        ],
        out_shape=[
            jax.ShapeDtypeStruct((_REMAIN, _S, 128), patches.dtype),
            jax.ShapeDtypeStruct((_REMAIN, 4, _B), jnp.int32),
        ],
    )(p)
    idx = idx.reshape(_T, _B)
    return (out.reshape(_REMAIN, _B, _C), idx, idx)


# speedup vs baseline: 4.8757x; 1.0521x over previous
import jax
import jax.numpy as jnp
from jax.experimental import pallas as pl

_T = 256
_B = 1024
_C = 192
_REMAIN = 64
_S = _B * _C // 128
_BT = 8
_STEPS = _REMAIN // _BT
_IDX_ROWS = _T // _STEPS


def _shuffle_kernel(p_ref, out_ref, idx_ref):
    i = pl.program_id(0)
    for k in range(_BT):
        out_ref[k, :, :] = p_ref[_BT - 1 - k, :, :]
    row = i * _IDX_ROWS + jax.lax.broadcasted_iota(
        jnp.int32, (1, _IDX_ROWS, _B), 1)
    idx_ref[...] = (_T - 1) - row


def kernel(patches):
    p = patches.reshape(_T, _S, 128)
    out, idx = pl.pallas_call(
        _shuffle_kernel,
        grid=(_STEPS,),
        in_specs=[
            pl.BlockSpec((_BT, _S, 128), lambda i: (_T // _BT - 1 - i, 0, 0)),
        ],
        out_specs=[
            pl.BlockSpec((_BT, _S, 128), lambda i: (i, 0, 0)),
            pl.BlockSpec((1, _IDX_ROWS, _B), lambda i: (i, 0, 0)),
        ],
        out_shape=[
            jax.ShapeDtypeStruct((_REMAIN, _S, 128), patches.dtype),
            jax.ShapeDtypeStruct((_STEPS, _IDX_ROWS, _B), jnp.int32),
        ],
    )(p)
    idx = idx.reshape(_T, _B)
    return (out.reshape(_REMAIN, _B, _C), idx, idx)


# speedup vs baseline: 7.3963x; 1.5170x over previous
import jax
import jax.numpy as jnp
from jax.experimental import pallas as pl

_T = 256
_B = 1024
_C = 192
_REMAIN = 64
_BT = 8
_STEPS = _REMAIN // _BT
_IDX_ROWS = _T // _STEPS


def _shuffle_kernel(p_ref, out_ref, idx_ref):
    i = pl.program_id(0)
    for k in range(_BT):
        out_ref[k, :, :] = p_ref[_BT - 1 - k, :, :]
    row = i * _IDX_ROWS + jax.lax.broadcasted_iota(
        jnp.int32, (_IDX_ROWS, _B), 0)
    idx_ref[...] = (_T - 1) - row


def kernel(patches):
    out, idx = pl.pallas_call(
        _shuffle_kernel,
        grid=(_STEPS,),
        in_specs=[
            pl.BlockSpec((_BT, _B, _C), lambda i: (_T // _BT - 1 - i, 0, 0)),
        ],
        out_specs=[
            pl.BlockSpec((_BT, _B, _C), lambda i: (i, 0, 0)),
            pl.BlockSpec((_IDX_ROWS, _B), lambda i: (i, 0)),
        ],
        out_shape=[
            jax.ShapeDtypeStruct((_REMAIN, _B, _C), patches.dtype),
            jax.ShapeDtypeStruct((_T, _B), jnp.int32),
        ],
    )(patches)
    return (out, idx, idx)
